# Initial kernel scaffold; baseline (speedup 1.0000x reference)
import jax
import jax.numpy as jnp
from jax.experimental import pallas as pl


def kernel(x, pos_table, seg_table, W, b):
    raise NotImplementedError("write your pallas kernel here")


# speedup vs baseline: 2.4061x; 2.4061x over previous
import functools

import jax
import jax.numpy as jnp
from jax.experimental import pallas as pl

CLS_TOKEN = 1.0
SEP_TOKEN = 2.0


def _body(x_ref, pos_ref, seg_ref, w_ref, b_ref, out_ref):
    seg0 = seg_ref[0:1, :]
    wb = w_ref[:].astype(jnp.bfloat16)
    h = x_ref.shape[-1]
    cls_row = jnp.full((1, h), CLS_TOKEN, dtype=jnp.float32)
    sep_row = jnp.full((1, h), SEP_TOKEN, dtype=jnp.float32)
    tokens = jnp.concatenate([cls_row, x_ref[0], sep_row], axis=0)
    vis = tokens + pos_ref[:] + seg0
    acc = jnp.dot(vis.astype(jnp.bfloat16), wb,
                  preferred_element_type=jnp.float32)
    out_ref[0] = acc + b_ref[:]


@jax.jit
def kernel(x, pos_table, seg_table, W, b):
    batch, sig_len, hid = x.shape
    emb = W.shape[1]
    n_rows = sig_len + 2
    b2 = b.reshape(1, emb)
    out = pl.pallas_call(
        _body,
        grid=(batch,),
        in_specs=[
            pl.BlockSpec((1, sig_len, hid), lambda i: (i, 0, 0)),
            pl.BlockSpec((n_rows, hid), lambda i: (0, 0)),
            pl.BlockSpec((2, hid), lambda i: (0, 0)),
            pl.BlockSpec((hid, emb), lambda i: (0, 0)),
            pl.BlockSpec((1, emb), lambda i: (0, 0)),
        ],
        out_specs=pl.BlockSpec((1, n_rows, emb), lambda i: (i, 0, 0)),
        out_shape=jax.ShapeDtypeStruct((batch, n_rows, emb), jnp.float32),
    )(x, pos_table, seg_table, W, b2)
    return out
